# baseline (speedup 1.0000x reference)
import functools

import jax
import jax.numpy as jnp
from jax import lax
from jax.experimental import pallas as pl
from jax.experimental.pallas import tpu as pltpu
from jax.experimental.pallas import tpu_sc as plsc

N = 320000
D = 128
B = 16
NC = 2
NS = 16
NW = NC * NS

RW = 6400
N_SC = NW * RW
CH = 200
NCHUNK = RW // CH
NG = D // 16

BLK = 3200
OFF = N_SC // BLK
NBLK = (N - N_SC) // BLK


def _sc_body(x_hbm, ed2_hbm, out_hbm, buf0, buf1, ed2_v, acc_v, sem0, sem1):
    cid = lax.axis_index("c")
    sid = lax.axis_index("s")
    wid = cid * NS + sid
    lo = wid * RW

    pltpu.sync_copy(ed2_hbm, ed2_v)
    ev_lo = ed2_v[0, :]
    ev_hi = ed2_v[1, :]

    zeros16 = jnp.zeros((16,), jnp.float32)
    for s in range(B):
        for g in range(NG):
            acc_v[s, pl.ds(16 * g, 16)] = zeros16

    bufs = (buf0, buf1)
    sems = (sem0, sem1)
    for b in range(2):
        pltpu.make_async_copy(
            x_hbm.at[pl.ds(lo + b * CH, CH)], bufs[b], sems[b]).start()

    iota = lax.iota(jnp.int32, 16)
    ones16 = jnp.ones((16,), jnp.int32)
    zeros16i = jnp.zeros((16,), jnp.int32)

    def seg_of(r):
        rv = jnp.full((16,), r, jnp.int32)
        return jnp.sum(jnp.where(ev_hi <= rv, ones16, zeros16i))

    def extract(vec, s):
        sv = jnp.full((16,), s, jnp.int32)
        return jnp.sum(jnp.where(iota == sv, vec, zeros16i))

    def process(buf, a):
        s_lo = seg_of(a)
        s_hi = seg_of(a + (CH - 1))

        def seg_body(s, _):
            rs = jnp.maximum(a, extract(ev_lo, s)) - a
            re = jnp.minimum(a + CH, extract(ev_hi, s)) - a
            init = tuple(jnp.zeros((16,), jnp.float32) for _ in range(NG))

            @plsc.parallel_loop(rs, re, 1, unroll=4, carry=init)
            def part(r, carry):
                return tuple(carry[g] + buf[r, pl.ds(16 * g, 16)]
                             for g in range(NG))

            for g in range(NG):
                acc_v[s, pl.ds(16 * g, 16)] = (
                    acc_v[s, pl.ds(16 * g, 16)] + part[g])
            return 0

        lax.fori_loop(s_lo, s_hi + 1, seg_body, 0)

    def chunk_pair(k, _):
        for b in range(2):
            c = 2 * k + b
            a = lo + c * CH
            pltpu.make_async_copy(
                x_hbm.at[pl.ds(a, CH)], bufs[b], sems[b]).wait()
            process(bufs[b], a)

            @pl.when(c + 2 < NCHUNK)
            def _prefetch():
                pltpu.make_async_copy(
                    x_hbm.at[pl.ds(a + 2 * CH, CH)], bufs[b], sems[b]).start()
        return 0

    lax.fori_loop(0, NCHUNK // 2, chunk_pair, 0)

    pltpu.sync_copy(acc_v, out_hbm.at[wid])


def _tc_seg_body(ed_ref, x_ref, o_ref):
    g = pl.program_id(0)
    a = N_SC + g * BLK

    @pl.when(g == 0)
    def _init():
        o_ref[...] = jnp.zeros_like(o_ref)

    rows = jax.lax.broadcasted_iota(jnp.int32, (1, BLK), 1) + a
    lo = jnp.concatenate(
        [ed_ref[0, s].reshape(1, 1) for s in range(B)], axis=0)
    hi = jnp.concatenate(
        [ed_ref[1, s].reshape(1, 1) for s in range(B)], axis=0)
    e = ((lo <= rows) & (rows < hi)).astype(jnp.float32)
    o_ref[...] += jax.lax.dot_general(
        e, x_ref[...], (((1,), (0,)), ((), ())),
        preferred_element_type=jnp.float32)


def _tc_combine_body(psc_ref, ptc_ref, w_ref, o_ref):
    xm = jnp.sum(psc_ref[...], axis=0) + ptc_ref[...]
    o_ref[...] = lax.dot_general(
        xm, w_ref[...], (((1,), (1,)), ((), ())),
        preferred_element_type=jnp.float32)


@jax.jit
def kernel(x, edge_slices, W):
    es = edge_slices.astype(jnp.int32)
    ed2 = jnp.stack([es[:B], es[1:B + 1]])

    sc = pl.kernel(
        _sc_body,
        out_type=jax.ShapeDtypeStruct((NW, B, D), jnp.float32),
        mesh=plsc.VectorSubcoreMesh(core_axis_name="c", subcore_axis_name="s",
                                    num_cores=NC, num_subcores=NS),
        compiler_params=pltpu.CompilerParams(needs_layout_passes=False),
        scratch_types=[
            pltpu.VMEM((CH, D), jnp.float32),
            pltpu.VMEM((CH, D), jnp.float32),
            pltpu.VMEM((2, 16), jnp.int32),
            pltpu.VMEM((B, D), jnp.float32),
            pltpu.SemaphoreType.DMA,
            pltpu.SemaphoreType.DMA,
        ],
    )
    partials_sc = sc(x, ed2)

    tc_seg = pl.pallas_call(
        _tc_seg_body,
        grid=(NBLK,),
        in_specs=[
            pl.BlockSpec(memory_space=pltpu.SMEM),
            pl.BlockSpec((BLK, D), lambda g: (OFF + g, 0)),
        ],
        out_specs=pl.BlockSpec((B, D), lambda g: (0, 0)),
        out_shape=jax.ShapeDtypeStruct((B, D), jnp.float32),
        compiler_params=pltpu.CompilerParams(
            dimension_semantics=("arbitrary",)),
    )
    partial_tc = tc_seg(ed2, x)

    out = pl.pallas_call(
        _tc_combine_body,
        out_shape=jax.ShapeDtypeStruct((B, D), jnp.float32),
    )(partials_sc, partial_tc, W)
    return out


# speedup vs baseline: 1.0231x; 1.0231x over previous
import functools

import jax
import jax.numpy as jnp
from jax import lax
from jax.experimental import pallas as pl
from jax.experimental.pallas import tpu as pltpu
from jax.experimental.pallas import tpu_sc as plsc

N = 320000
D = 128
B = 16
NC = 2
NS = 16
NW = NC * NS

RW = 6000
N_SC = NW * RW
CH = 200
NCHUNK = RW // CH
NG = D // 16

BLK = 3200
OFF = N_SC // BLK
NBLK = (N - N_SC) // BLK


def _sc_body(x_hbm, ed2_hbm, out_hbm, buf0, buf1, ed2_v, acc_v, sem0, sem1):
    cid = lax.axis_index("c")
    sid = lax.axis_index("s")
    wid = cid * NS + sid
    lo = wid * RW

    pltpu.sync_copy(ed2_hbm, ed2_v)
    ev_lo = ed2_v[0, :]
    ev_hi = ed2_v[1, :]

    zeros16 = jnp.zeros((16,), jnp.float32)
    for s in range(B):
        for g in range(NG):
            acc_v[s, pl.ds(16 * g, 16)] = zeros16

    bufs = (buf0, buf1)
    sems = (sem0, sem1)
    for b in range(2):
        pltpu.make_async_copy(
            x_hbm.at[pl.ds(lo + b * CH, CH)], bufs[b], sems[b]).start()

    iota = lax.iota(jnp.int32, 16)
    ones16 = jnp.ones((16,), jnp.int32)
    zeros16i = jnp.zeros((16,), jnp.int32)

    def seg_of(r):
        rv = jnp.full((16,), r, jnp.int32)
        return jnp.sum(jnp.where(ev_hi <= rv, ones16, zeros16i))

    def extract(vec, s):
        sv = jnp.full((16,), s, jnp.int32)
        return jnp.sum(jnp.where(iota == sv, vec, zeros16i))

    def process(buf, a):
        s_lo = seg_of(a)
        s_hi = seg_of(a + (CH - 1))

        def seg_body(s, _):
            rs = jnp.maximum(a, extract(ev_lo, s)) - a
            re = jnp.minimum(a + CH, extract(ev_hi, s)) - a
            init = tuple(jnp.zeros((16,), jnp.float32) for _ in range(NG))

            @plsc.parallel_loop(rs, re, 1, unroll=4, carry=init)
            def part(r, carry):
                return tuple(carry[g] + buf[r, pl.ds(16 * g, 16)]
                             for g in range(NG))

            for g in range(NG):
                acc_v[s, pl.ds(16 * g, 16)] = (
                    acc_v[s, pl.ds(16 * g, 16)] + part[g])
            return 0

        lax.fori_loop(s_lo, s_hi + 1, seg_body, 0)

    def chunk_pair(k, _):
        for b in range(2):
            c = 2 * k + b
            a = lo + c * CH
            pltpu.make_async_copy(
                x_hbm.at[pl.ds(a, CH)], bufs[b], sems[b]).wait()
            process(bufs[b], a)

            @pl.when(c + 2 < NCHUNK)
            def _prefetch():
                pltpu.make_async_copy(
                    x_hbm.at[pl.ds(a + 2 * CH, CH)], bufs[b], sems[b]).start()
        return 0

    lax.fori_loop(0, NCHUNK // 2, chunk_pair, 0)

    pltpu.sync_copy(acc_v, out_hbm.at[wid])


def _tc_seg_body(ed_ref, x_ref, o_ref):
    g = pl.program_id(0)
    a = N_SC + g * BLK

    @pl.when(g == 0)
    def _init():
        o_ref[...] = jnp.zeros_like(o_ref)

    rows = jax.lax.broadcasted_iota(jnp.int32, (1, BLK), 1) + a
    lo = jnp.concatenate(
        [ed_ref[0, s].reshape(1, 1) for s in range(B)], axis=0)
    hi = jnp.concatenate(
        [ed_ref[1, s].reshape(1, 1) for s in range(B)], axis=0)
    e = ((lo <= rows) & (rows < hi)).astype(jnp.float32)
    o_ref[...] += jax.lax.dot_general(
        e, x_ref[...], (((1,), (0,)), ((), ())),
        preferred_element_type=jnp.float32)


def _tc_combine_body(psc_ref, ptc_ref, w_ref, o_ref):
    xm = jnp.sum(psc_ref[...], axis=0) + ptc_ref[...]
    o_ref[...] = lax.dot_general(
        xm, w_ref[...], (((1,), (1,)), ((), ())),
        preferred_element_type=jnp.float32)


@jax.jit
def kernel(x, edge_slices, W):
    es = edge_slices.astype(jnp.int32)
    ed2 = jnp.stack([es[:B], es[1:B + 1]])

    sc = pl.kernel(
        _sc_body,
        out_type=jax.ShapeDtypeStruct((NW, B, D), jnp.float32),
        mesh=plsc.VectorSubcoreMesh(core_axis_name="c", subcore_axis_name="s",
                                    num_cores=NC, num_subcores=NS),
        compiler_params=pltpu.CompilerParams(needs_layout_passes=False),
        scratch_types=[
            pltpu.VMEM((CH, D), jnp.float32),
            pltpu.VMEM((CH, D), jnp.float32),
            pltpu.VMEM((2, 16), jnp.int32),
            pltpu.VMEM((B, D), jnp.float32),
            pltpu.SemaphoreType.DMA,
            pltpu.SemaphoreType.DMA,
        ],
    )
    partials_sc = sc(x, ed2)

    tc_seg = pl.pallas_call(
        _tc_seg_body,
        grid=(NBLK,),
        in_specs=[
            pl.BlockSpec(memory_space=pltpu.SMEM),
            pl.BlockSpec((BLK, D), lambda g: (OFF + g, 0)),
        ],
        out_specs=pl.BlockSpec((B, D), lambda g: (0, 0)),
        out_shape=jax.ShapeDtypeStruct((B, D), jnp.float32),
        compiler_params=pltpu.CompilerParams(
            dimension_semantics=("arbitrary",)),
    )
    partial_tc = tc_seg(ed2, x)

    out = pl.pallas_call(
        _tc_combine_body,
        out_shape=jax.ShapeDtypeStruct((B, D), jnp.float32),
    )(partials_sc, partial_tc, W)
    return out


# speedup vs baseline: 1.0577x; 1.0338x over previous
import functools

import jax
import jax.numpy as jnp
from jax import lax
from jax.experimental import pallas as pl
from jax.experimental.pallas import tpu as pltpu
from jax.experimental.pallas import tpu_sc as plsc

N = 320000
D = 128
B = 16
NC = 2
NS = 16
NW = NC * NS

RW = 5600
N_SC = NW * RW
CH = 200
NCHUNK = RW // CH
NG = D // 16

BLK = 3200
OFF = N_SC // BLK
NBLK = (N - N_SC) // BLK


def _sc_body(x_hbm, ed2_hbm, out_hbm, buf0, buf1, ed2_v, acc_v, sem0, sem1):
    cid = lax.axis_index("c")
    sid = lax.axis_index("s")
    wid = cid * NS + sid
    lo = wid * RW

    pltpu.sync_copy(ed2_hbm, ed2_v)
    ev_lo = ed2_v[0, :]
    ev_hi = ed2_v[1, :]

    zeros16 = jnp.zeros((16,), jnp.float32)
    for s in range(B):
        for g in range(NG):
            acc_v[s, pl.ds(16 * g, 16)] = zeros16

    bufs = (buf0, buf1)
    sems = (sem0, sem1)
    for b in range(2):
        pltpu.make_async_copy(
            x_hbm.at[pl.ds(lo + b * CH, CH)], bufs[b], sems[b]).start()

    iota = lax.iota(jnp.int32, 16)
    ones16 = jnp.ones((16,), jnp.int32)
    zeros16i = jnp.zeros((16,), jnp.int32)

    def seg_of(r):
        rv = jnp.full((16,), r, jnp.int32)
        return jnp.sum(jnp.where(ev_hi <= rv, ones16, zeros16i))

    def extract(vec, s):
        sv = jnp.full((16,), s, jnp.int32)
        return jnp.sum(jnp.where(iota == sv, vec, zeros16i))

    def process(buf, a):
        s_lo = seg_of(a)
        s_hi = seg_of(a + (CH - 1))

        def seg_body(s, _):
            rs = jnp.maximum(a, extract(ev_lo, s)) - a
            re = jnp.minimum(a + CH, extract(ev_hi, s)) - a
            init = tuple(jnp.zeros((16,), jnp.float32) for _ in range(NG))

            @plsc.parallel_loop(rs, re, 1, unroll=4, carry=init)
            def part(r, carry):
                return tuple(carry[g] + buf[r, pl.ds(16 * g, 16)]
                             for g in range(NG))

            for g in range(NG):
                acc_v[s, pl.ds(16 * g, 16)] = (
                    acc_v[s, pl.ds(16 * g, 16)] + part[g])
            return 0

        lax.fori_loop(s_lo, s_hi + 1, seg_body, 0)

    def chunk_pair(k, _):
        for b in range(2):
            c = 2 * k + b
            a = lo + c * CH
            pltpu.make_async_copy(
                x_hbm.at[pl.ds(a, CH)], bufs[b], sems[b]).wait()
            process(bufs[b], a)

            @pl.when(c + 2 < NCHUNK)
            def _prefetch():
                pltpu.make_async_copy(
                    x_hbm.at[pl.ds(a + 2 * CH, CH)], bufs[b], sems[b]).start()
        return 0

    lax.fori_loop(0, NCHUNK // 2, chunk_pair, 0)

    pltpu.sync_copy(acc_v, out_hbm.at[wid])


def _tc_seg_body(ed_ref, x_ref, o_ref):
    g = pl.program_id(0)
    a = N_SC + g * BLK

    @pl.when(g == 0)
    def _init():
        o_ref[...] = jnp.zeros_like(o_ref)

    rows = jax.lax.broadcasted_iota(jnp.int32, (1, BLK), 1) + a
    lo = jnp.concatenate(
        [ed_ref[0, s].reshape(1, 1) for s in range(B)], axis=0)
    hi = jnp.concatenate(
        [ed_ref[1, s].reshape(1, 1) for s in range(B)], axis=0)
    e = ((lo <= rows) & (rows < hi)).astype(jnp.float32)
    o_ref[...] += jax.lax.dot_general(
        e, x_ref[...], (((1,), (0,)), ((), ())),
        preferred_element_type=jnp.float32)


def _tc_combine_body(psc_ref, ptc_ref, w_ref, o_ref):
    xm = jnp.sum(psc_ref[...], axis=0) + ptc_ref[...]
    o_ref[...] = lax.dot_general(
        xm, w_ref[...], (((1,), (1,)), ((), ())),
        preferred_element_type=jnp.float32)


@jax.jit
def kernel(x, edge_slices, W):
    es = edge_slices.astype(jnp.int32)
    ed2 = jnp.stack([es[:B], es[1:B + 1]])

    sc = pl.kernel(
        _sc_body,
        out_type=jax.ShapeDtypeStruct((NW, B, D), jnp.float32),
        mesh=plsc.VectorSubcoreMesh(core_axis_name="c", subcore_axis_name="s",
                                    num_cores=NC, num_subcores=NS),
        compiler_params=pltpu.CompilerParams(needs_layout_passes=False),
        scratch_types=[
            pltpu.VMEM((CH, D), jnp.float32),
            pltpu.VMEM((CH, D), jnp.float32),
            pltpu.VMEM((2, 16), jnp.int32),
            pltpu.VMEM((B, D), jnp.float32),
            pltpu.SemaphoreType.DMA,
            pltpu.SemaphoreType.DMA,
        ],
    )
    partials_sc = sc(x, ed2)

    tc_seg = pl.pallas_call(
        _tc_seg_body,
        grid=(NBLK,),
        in_specs=[
            pl.BlockSpec(memory_space=pltpu.SMEM),
            pl.BlockSpec((BLK, D), lambda g: (OFF + g, 0)),
        ],
        out_specs=pl.BlockSpec((B, D), lambda g: (0, 0)),
        out_shape=jax.ShapeDtypeStruct((B, D), jnp.float32),
        compiler_params=pltpu.CompilerParams(
            dimension_semantics=("arbitrary",)),
    )
    partial_tc = tc_seg(ed2, x)

    out = pl.pallas_call(
        _tc_combine_body,
        out_shape=jax.ShapeDtypeStruct((B, D), jnp.float32),
    )(partials_sc, partial_tc, W)
    return out
